# baseline (speedup 1.0000x reference)
import functools

import jax
import jax.numpy as jnp
from jax import lax
from jax.experimental import pallas as pl
from jax.experimental.pallas import tpu as pltpu
from jax.experimental.pallas import tpu_sc as plsc

N, E, F, H, C = 10000, 320000, 128, 128, 40
NC, NS = 2, 16
CHUNK = 80
DEG_W = 16
ROW_BLK = 1000
GRID = N // ROW_BLK


def _sc_mesh():
    return plsc.VectorSubcoreMesh(core_axis_name="c", subcore_axis_name="s")


N_PAD = 10240
HROWS = N_PAD // 128


def _sc_degree(dst, idrows, zeros_hist):
    per_sc = E // NC
    per_tile = per_sc // NS
    n_groups = per_tile // 16

    @functools.partial(
        pl.kernel,
        out_type=jax.ShapeDtypeStruct((NC, HROWS, 128), jnp.float32),
        mesh=_sc_mesh(),
        compiler_params=pltpu.CompilerParams(needs_layout_passes=False),
        scratch_types=[
            pltpu.VMEM((per_tile,), jnp.int32),
            pltpu.VMEM((HROWS,), jnp.int32),
            pltpu.VMEM((HROWS, 128), jnp.float32),
            pltpu.VMEM_SHARED((HROWS, 128), jnp.float32),
        ],
    )
    def kdeg(dst_hbm, idr_hbm, z_hbm, out_hbm, didx, idr_v, hist, acc):
        cid = lax.axis_index("c")
        sid = lax.axis_index("s")
        pltpu.sync_copy(z_hbm, hist)
        pltpu.sync_copy(idr_hbm, idr_v)
        base0 = cid * per_sc + sid * per_tile
        pltpu.sync_copy(dst_hbm.at[pl.ds(base0, per_tile)], didx)

        @pl.when(sid == 0)
        def _zero():
            pltpu.sync_copy(z_hbm, acc)

        plsc.subcore_barrier()
        ones = jnp.full((16,), 1.0, jnp.float32)

        def body(i, carry):
            v = didx[pl.ds(i * 16, 16)]
            plsc.addupdate_scatter(hist, [v >> 7, v & 127], ones)
            return carry

        lax.fori_loop(0, n_groups, body, 0)
        pltpu.sync_copy(hist, acc.at[idr_v], add=True)
        plsc.subcore_barrier()

        @pl.when(sid == 0)
        def _out():
            pltpu.sync_copy(acc, out_hbm.at[cid])

    return kdeg(dst, idrows, zeros_hist)


NBUF = 3


def _sc_aggregate(g, src3, dst3, zeros_stripe):
    per_sc = E // NC
    per_tile = per_sc // NS
    n_chunks = per_tile // CHUNK
    n_outer = n_chunks // NBUF
    n_tail = n_chunks - n_outer * NBUF
    stripe = ROW_BLK

    @functools.partial(
        pl.kernel,
        out_type=jax.ShapeDtypeStruct((NC, N, H), jnp.float32),
        mesh=_sc_mesh(),
        scratch_types=[
            pltpu.VMEM((per_tile,), jnp.int32),
            pltpu.VMEM((per_tile,), jnp.int32),
            pltpu.VMEM((NBUF, CHUNK, H), jnp.float32),
            pltpu.VMEM_SHARED((N, H), jnp.float32),
        ]
        + [pltpu.SemaphoreType.DMA] * NBUF,
    )
    def kagg(g_hbm, src_hbm, dst_hbm, z_hbm, out_hbm, sidx, didx, rows, acc, *sems):
        cid = lax.axis_index("c")
        sid = lax.axis_index("s")
        base0 = (cid * NS + sid) * per_tile
        pltpu.sync_copy(src_hbm.at[pl.ds(base0, per_tile)], sidx)
        pltpu.sync_copy(dst_hbm.at[pl.ds(base0, per_tile)], didx)

        @pl.when(sid < GRID)
        def _zero():
            pltpu.sync_copy(z_hbm, acc.at[pl.ds(sid * stripe, stripe)])

        plsc.subcore_barrier()

        def _chunk(j):
            return pl.ds(j * CHUNK, CHUNK)

        for b in range(NBUF):
            pltpu.async_copy(g_hbm.at[sidx.at[_chunk(b)]], rows.at[b], sems[b])

        def body(t, carry):
            j0 = t * NBUF
            for b in range(NBUF):
                j = j0 + b
                pltpu.make_async_copy(
                    g_hbm.at[sidx.at[_chunk(b)]], rows.at[b], sems[b]
                ).wait()
                pltpu.sync_copy(rows.at[b], acc.at[didx.at[_chunk(j)]], add=True)

                @pl.when(j + NBUF < n_chunks)
                def _refire():
                    pltpu.async_copy(
                        g_hbm.at[sidx.at[_chunk(j + NBUF)]], rows.at[b], sems[b]
                    )

            return carry

        lax.fori_loop(0, n_outer, body, 0)
        for b in range(n_tail):
            j = n_outer * NBUF + b
            pltpu.make_async_copy(
                g_hbm.at[sidx.at[_chunk(b)]], rows.at[b], sems[b]
            ).wait()
            pltpu.sync_copy(rows.at[b], acc.at[didx.at[_chunk(j)]], add=True)
        plsc.subcore_barrier()

        @pl.when(sid < GRID)
        def _out():
            pltpu.sync_copy(
                acc.at[pl.ds(sid * stripe, stripe)],
                out_hbm.at[cid].at[pl.ds(sid * stripe, stripe)],
            )

    return kagg(g, src3, dst3, zeros_stripe)


def _dinv_block(d0_ref, d1_ref):
    return lax.rsqrt(d0_ref[...] + d1_ref[...] + 1.0)


def _tc_g1_body(x_ref, w_ref, d0_ref, d1_ref, o_ref):
    dinv = _dinv_block(d0_ref, d1_ref)
    o_ref[...] = dinv * jnp.dot(
        x_ref[...], w_ref[...], preferred_element_type=jnp.float32
    )


def _tc_mid_body(agg_ref, g_ref, d0_ref, d1_ref, b_ref, w_ref, o_ref):
    dinv = _dinv_block(d0_ref, d1_ref)
    h = dinv * (agg_ref[0] + agg_ref[1] + g_ref[...]) + b_ref[...]
    h = jnp.maximum(h, 0.0)
    o_ref[...] = dinv * jnp.dot(h, w_ref[...], preferred_element_type=jnp.float32)


def _tc_final_body(agg_ref, g_ref, d0_ref, d1_ref, b_ref, wc_ref, bc_ref, o_ref):
    dinv = _dinv_block(d0_ref, d1_ref)
    h = dinv * (agg_ref[0] + agg_ref[1] + g_ref[...]) + b_ref[...]
    o_ref[...] = (
        jnp.dot(h, wc_ref[...], preferred_element_type=jnp.float32) + bc_ref[...]
    )


def _row_spec(width):
    return pl.BlockSpec((ROW_BLK, width), lambda i: (i, 0))


_D_SPEC = pl.BlockSpec((ROW_BLK, 1), lambda i: (i, 0))
_AGG_SPEC = pl.BlockSpec((NC, ROW_BLK, H), lambda i: (0, i, 0))
_FULL_W = pl.BlockSpec((F, H), lambda i: (0, 0))
_BIAS_SPEC = pl.BlockSpec((1, H), lambda i: (0, 0))


def _tc_g1(x, W1, d0, d1):
    return pl.pallas_call(
        _tc_g1_body,
        grid=(GRID,),
        in_specs=[_row_spec(F), _FULL_W, _D_SPEC, _D_SPEC],
        out_specs=_row_spec(H),
        out_shape=jax.ShapeDtypeStruct((N, H), jnp.float32),
    )(x, W1, d0, d1)


def _tc_mid(aggp, g1, d0, d1, b1, W2):
    return pl.pallas_call(
        _tc_mid_body,
        grid=(GRID,),
        in_specs=[_AGG_SPEC, _row_spec(H), _D_SPEC, _D_SPEC, _BIAS_SPEC, _FULL_W],
        out_specs=_row_spec(H),
        out_shape=jax.ShapeDtypeStruct((N, H), jnp.float32),
    )(aggp, g1, d0, d1, b1, W2)


def _tc_final(aggp, g2, d0, d1, b2, Wc, bc):
    return pl.pallas_call(
        _tc_final_body,
        grid=(GRID,),
        in_specs=[
            _AGG_SPEC,
            _row_spec(H),
            _D_SPEC,
            _D_SPEC,
            _BIAS_SPEC,
            pl.BlockSpec((H, C), lambda i: (0, 0)),
            pl.BlockSpec((1, C), lambda i: (0, 0)),
        ],
        out_specs=pl.BlockSpec((ROW_BLK, C), lambda i: (i, 0)),
        out_shape=jax.ShapeDtypeStruct((N, C), jnp.float32),
    )(aggp, g2, d0, d1, b2, Wc, bc)


def kernel(x, edge_index, W1, b1, W2, b2, Wc, bc):
    src = jnp.asarray(edge_index[0], jnp.int32)
    dst = jnp.asarray(edge_index[1], jnp.int32)

    zeros_h = jnp.zeros((ROW_BLK, H), jnp.float32)
    zeros_hist = jnp.zeros((HROWS, 128), jnp.float32)
    idrows = jnp.arange(HROWS, dtype=jnp.int32)

    b1r = b1.reshape(1, H)
    b2r = b2.reshape(1, H)

    degp = _sc_degree(dst, idrows, zeros_hist)
    degf = degp.reshape(NC, N_PAD)[:, :N]
    d0 = degf[0].reshape(N, 1)
    d1 = degf[1].reshape(N, 1)

    g1 = _tc_g1(x, W1, d0, d1)
    agg1 = _sc_aggregate(g1, src, dst, zeros_h)
    g2 = _tc_mid(agg1, g1, d0, d1, b1r, W2)
    agg2 = _sc_aggregate(g2, src, dst, zeros_h)
    return _tc_final(agg2, g2, d0, d1, b2r, Wc, bc.reshape(1, C))


# speedup vs baseline: 1.0213x; 1.0213x over previous
import functools

import jax
import jax.numpy as jnp
from jax import lax
from jax.experimental import pallas as pl
from jax.experimental.pallas import tpu as pltpu
from jax.experimental.pallas import tpu_sc as plsc

N, E, F, H, C = 10000, 320000, 128, 128, 40
NC, NS = 2, 16
CHUNK = 80
DEG_W = 16
ROW_BLK = 2000
GRID = N // ROW_BLK


def _sc_mesh():
    return plsc.VectorSubcoreMesh(core_axis_name="c", subcore_axis_name="s")


N_PAD = 10240
HROWS = N_PAD // 128


def _sc_degree(dst, idrows, zeros_hist):
    per_sc = E // NC
    per_tile = per_sc // NS
    n_groups = per_tile // 16

    @functools.partial(
        pl.kernel,
        out_type=jax.ShapeDtypeStruct((NC, HROWS, 128), jnp.float32),
        mesh=_sc_mesh(),
        compiler_params=pltpu.CompilerParams(needs_layout_passes=False),
        scratch_types=[
            pltpu.VMEM((per_tile,), jnp.int32),
            pltpu.VMEM((HROWS,), jnp.int32),
            pltpu.VMEM((HROWS, 128), jnp.float32),
            pltpu.VMEM_SHARED((HROWS, 128), jnp.float32),
        ],
    )
    def kdeg(dst_hbm, idr_hbm, z_hbm, out_hbm, didx, idr_v, hist, acc):
        cid = lax.axis_index("c")
        sid = lax.axis_index("s")
        pltpu.sync_copy(z_hbm, hist)
        pltpu.sync_copy(idr_hbm, idr_v)
        base0 = cid * per_sc + sid * per_tile
        pltpu.sync_copy(dst_hbm.at[pl.ds(base0, per_tile)], didx)

        @pl.when(sid == 0)
        def _zero():
            pltpu.sync_copy(z_hbm, acc)

        plsc.subcore_barrier()
        ones = jnp.full((16,), 1.0, jnp.float32)

        def body(i, carry):
            v = didx[pl.ds(i * 16, 16)]
            plsc.addupdate_scatter(hist, [v >> 7, v & 127], ones)
            return carry

        lax.fori_loop(0, n_groups, body, 0)
        pltpu.sync_copy(hist, acc.at[idr_v], add=True)
        plsc.subcore_barrier()

        @pl.when(sid == 0)
        def _out():
            pltpu.sync_copy(acc, out_hbm.at[cid])

    return kdeg(dst, idrows, zeros_hist)


NBUF = 3


def _sc_aggregate(g, src3, dst3, zeros_stripe):
    per_sc = E // NC
    per_tile = per_sc // NS
    n_chunks = per_tile // CHUNK
    n_outer = n_chunks // NBUF
    n_tail = n_chunks - n_outer * NBUF
    stripe = ROW_BLK

    @functools.partial(
        pl.kernel,
        out_type=jax.ShapeDtypeStruct((NC, N, H), jnp.float32),
        mesh=_sc_mesh(),
        scratch_types=[
            pltpu.VMEM((per_tile,), jnp.int32),
            pltpu.VMEM((per_tile,), jnp.int32),
            pltpu.VMEM((NBUF, CHUNK, H), jnp.float32),
            pltpu.VMEM_SHARED((N, H), jnp.float32),
        ]
        + [pltpu.SemaphoreType.DMA] * NBUF,
    )
    def kagg(g_hbm, src_hbm, dst_hbm, z_hbm, out_hbm, sidx, didx, rows, acc, *sems):
        cid = lax.axis_index("c")
        sid = lax.axis_index("s")
        base0 = (cid * NS + sid) * per_tile
        pltpu.sync_copy(src_hbm.at[pl.ds(base0, per_tile)], sidx)
        pltpu.sync_copy(dst_hbm.at[pl.ds(base0, per_tile)], didx)

        @pl.when(sid < GRID)
        def _zero():
            pltpu.sync_copy(z_hbm, acc.at[pl.ds(sid * stripe, stripe)])

        plsc.subcore_barrier()

        def _chunk(j):
            return pl.ds(j * CHUNK, CHUNK)

        for b in range(NBUF):
            pltpu.async_copy(g_hbm.at[sidx.at[_chunk(b)]], rows.at[b], sems[b])

        def body(t, carry):
            j0 = t * NBUF
            for b in range(NBUF):
                j = j0 + b
                pltpu.make_async_copy(
                    g_hbm.at[sidx.at[_chunk(b)]], rows.at[b], sems[b]
                ).wait()
                pltpu.sync_copy(rows.at[b], acc.at[didx.at[_chunk(j)]], add=True)

                @pl.when(j + NBUF < n_chunks)
                def _refire():
                    pltpu.async_copy(
                        g_hbm.at[sidx.at[_chunk(j + NBUF)]], rows.at[b], sems[b]
                    )

            return carry

        lax.fori_loop(0, n_outer, body, 0)
        for b in range(n_tail):
            j = n_outer * NBUF + b
            pltpu.make_async_copy(
                g_hbm.at[sidx.at[_chunk(b)]], rows.at[b], sems[b]
            ).wait()
            pltpu.sync_copy(rows.at[b], acc.at[didx.at[_chunk(j)]], add=True)
        plsc.subcore_barrier()

        @pl.when(sid < GRID)
        def _out():
            pltpu.sync_copy(
                acc.at[pl.ds(sid * stripe, stripe)],
                out_hbm.at[cid].at[pl.ds(sid * stripe, stripe)],
            )

    return kagg(g, src3, dst3, zeros_stripe)


def _dinv_block(d0_ref, d1_ref):
    return lax.rsqrt(d0_ref[...] + d1_ref[...] + 1.0)


def _tc_g1_body(x_ref, w_ref, d0_ref, d1_ref, o_ref):
    dinv = _dinv_block(d0_ref, d1_ref)
    o_ref[...] = dinv * jnp.dot(
        x_ref[...], w_ref[...], preferred_element_type=jnp.float32
    )


def _tc_mid_body(agg_ref, g_ref, d0_ref, d1_ref, b_ref, w_ref, o_ref):
    dinv = _dinv_block(d0_ref, d1_ref)
    h = dinv * (agg_ref[0] + agg_ref[1] + g_ref[...]) + b_ref[...]
    h = jnp.maximum(h, 0.0)
    o_ref[...] = dinv * jnp.dot(h, w_ref[...], preferred_element_type=jnp.float32)


def _tc_final_body(agg_ref, g_ref, d0_ref, d1_ref, b_ref, wc_ref, bc_ref, o_ref):
    dinv = _dinv_block(d0_ref, d1_ref)
    h = dinv * (agg_ref[0] + agg_ref[1] + g_ref[...]) + b_ref[...]
    o_ref[...] = (
        jnp.dot(h, wc_ref[...], preferred_element_type=jnp.float32) + bc_ref[...]
    )


def _row_spec(width):
    return pl.BlockSpec((ROW_BLK, width), lambda i: (i, 0))


_D_SPEC = pl.BlockSpec((ROW_BLK, 1), lambda i: (i, 0))
_AGG_SPEC = pl.BlockSpec((NC, ROW_BLK, H), lambda i: (0, i, 0))
_FULL_W = pl.BlockSpec((F, H), lambda i: (0, 0))
_BIAS_SPEC = pl.BlockSpec((1, H), lambda i: (0, 0))


def _tc_g1(x, W1, d0, d1):
    return pl.pallas_call(
        _tc_g1_body,
        grid=(GRID,),
        in_specs=[_row_spec(F), _FULL_W, _D_SPEC, _D_SPEC],
        out_specs=_row_spec(H),
        out_shape=jax.ShapeDtypeStruct((N, H), jnp.float32),
    )(x, W1, d0, d1)


def _tc_mid(aggp, g1, d0, d1, b1, W2):
    return pl.pallas_call(
        _tc_mid_body,
        grid=(GRID,),
        in_specs=[_AGG_SPEC, _row_spec(H), _D_SPEC, _D_SPEC, _BIAS_SPEC, _FULL_W],
        out_specs=_row_spec(H),
        out_shape=jax.ShapeDtypeStruct((N, H), jnp.float32),
    )(aggp, g1, d0, d1, b1, W2)


def _tc_final(aggp, g2, d0, d1, b2, Wc, bc):
    return pl.pallas_call(
        _tc_final_body,
        grid=(GRID,),
        in_specs=[
            _AGG_SPEC,
            _row_spec(H),
            _D_SPEC,
            _D_SPEC,
            _BIAS_SPEC,
            pl.BlockSpec((H, C), lambda i: (0, 0)),
            pl.BlockSpec((1, C), lambda i: (0, 0)),
        ],
        out_specs=pl.BlockSpec((ROW_BLK, C), lambda i: (i, 0)),
        out_shape=jax.ShapeDtypeStruct((N, C), jnp.float32),
    )(aggp, g2, d0, d1, b2, Wc, bc)


def kernel(x, edge_index, W1, b1, W2, b2, Wc, bc):
    src = jnp.asarray(edge_index[0], jnp.int32)
    dst = jnp.asarray(edge_index[1], jnp.int32)

    zeros_h = jnp.zeros((ROW_BLK, H), jnp.float32)
    zeros_hist = jnp.zeros((HROWS, 128), jnp.float32)
    idrows = jnp.arange(HROWS, dtype=jnp.int32)

    b1r = b1.reshape(1, H)
    b2r = b2.reshape(1, H)

    degp = _sc_degree(dst, idrows, zeros_hist)
    degf = degp.reshape(NC, N_PAD)[:, :N]
    d0 = degf[0].reshape(N, 1)
    d1 = degf[1].reshape(N, 1)

    g1 = _tc_g1(x, W1, d0, d1)
    agg1 = _sc_aggregate(g1, src, dst, zeros_h)
    g2 = _tc_mid(agg1, g1, d0, d1, b1r, W2)
    agg2 = _sc_aggregate(g2, src, dst, zeros_h)
    return _tc_final(agg2, g2, d0, d1, b2r, Wc, bc.reshape(1, C))


# speedup vs baseline: 1.0220x; 1.0007x over previous
import functools

import jax
import jax.numpy as jnp
from jax import lax
from jax.experimental import pallas as pl
from jax.experimental.pallas import tpu as pltpu
from jax.experimental.pallas import tpu_sc as plsc

N, E, F, H, C = 10000, 320000, 128, 128, 40
NC, NS = 2, 16
CHUNK = 80
ROW_BLK = 2000
GRID = N // ROW_BLK


def _sc_mesh():
    return plsc.VectorSubcoreMesh(core_axis_name="c", subcore_axis_name="s")


N_PAD = 10240
HROWS = N_PAD // 128


def _sc_degree(dst, idrows, zeros_hist):
    per_sc = E // NC
    per_tile = per_sc // NS
    n_groups = per_tile // 16

    @functools.partial(
        pl.kernel,
        out_type=jax.ShapeDtypeStruct((NC, HROWS, 128), jnp.float32),
        mesh=_sc_mesh(),
        compiler_params=pltpu.CompilerParams(needs_layout_passes=False),
        scratch_types=[
            pltpu.VMEM((per_tile,), jnp.int32),
            pltpu.VMEM((HROWS,), jnp.int32),
            pltpu.VMEM((HROWS, 128), jnp.float32),
            pltpu.VMEM_SHARED((HROWS, 128), jnp.float32),
        ],
    )
    def kdeg(dst_hbm, idr_hbm, z_hbm, out_hbm, didx, idr_v, hist, acc):
        cid = lax.axis_index("c")
        sid = lax.axis_index("s")
        pltpu.sync_copy(z_hbm, hist)
        pltpu.sync_copy(idr_hbm, idr_v)
        base0 = cid * per_sc + sid * per_tile
        pltpu.sync_copy(dst_hbm.at[pl.ds(base0, per_tile)], didx)

        @pl.when(sid == 0)
        def _zero():
            pltpu.sync_copy(z_hbm, acc)

        plsc.subcore_barrier()
        ones = jnp.full((16,), 1.0, jnp.float32)

        def body(i, carry):
            v = didx[pl.ds(i * 16, 16)]
            plsc.addupdate_scatter(hist, [v >> 7, v & 127], ones)
            return carry

        lax.fori_loop(0, n_groups, body, 0)
        pltpu.sync_copy(hist, acc.at[idr_v], add=True)
        plsc.subcore_barrier()

        @pl.when(sid == 0)
        def _out():
            pltpu.sync_copy(acc, out_hbm.at[cid])

    return kdeg(dst, idrows, zeros_hist)


NBUF = 3


def _sc_aggregate(g, src3, dst3, zeros_stripe):
    per_sc = E // NC
    per_tile = per_sc // NS
    n_chunks = per_tile // CHUNK
    n_outer = n_chunks // NBUF
    n_tail = n_chunks - n_outer * NBUF
    stripe = ROW_BLK

    @functools.partial(
        pl.kernel,
        out_type=jax.ShapeDtypeStruct((NC, N, H), jnp.float32),
        mesh=_sc_mesh(),
        scratch_types=[
            pltpu.VMEM((per_tile,), jnp.int32),
            pltpu.VMEM((per_tile,), jnp.int32),
            pltpu.VMEM((NBUF, CHUNK, H), jnp.float32),
            pltpu.VMEM_SHARED((N, H), jnp.float32),
        ]
        + [pltpu.SemaphoreType.DMA] * NBUF,
    )
    def kagg(g_hbm, src_hbm, dst_hbm, z_hbm, out_hbm, sidx, didx, rows, acc, *sems):
        cid = lax.axis_index("c")
        sid = lax.axis_index("s")
        base0 = (cid * NS + sid) * per_tile
        pltpu.sync_copy(src_hbm.at[pl.ds(base0, per_tile)], sidx)
        pltpu.sync_copy(dst_hbm.at[pl.ds(base0, per_tile)], didx)

        @pl.when(sid < GRID)
        def _zero():
            pltpu.sync_copy(z_hbm, acc.at[pl.ds(sid * stripe, stripe)])

        plsc.subcore_barrier()

        def _chunk(j):
            return pl.ds(j * CHUNK, CHUNK)

        for b in range(NBUF):
            pltpu.async_copy(g_hbm.at[sidx.at[_chunk(b)]], rows.at[b], sems[b])

        def body(t, carry):
            j0 = t * NBUF
            for b in range(NBUF):
                j = j0 + b
                pltpu.make_async_copy(
                    g_hbm.at[sidx.at[_chunk(b)]], rows.at[b], sems[b]
                ).wait()
                pltpu.sync_copy(rows.at[b], acc.at[didx.at[_chunk(j)]], add=True)

                @pl.when(j + NBUF < n_chunks)
                def _refire():
                    pltpu.async_copy(
                        g_hbm.at[sidx.at[_chunk(j + NBUF)]], rows.at[b], sems[b]
                    )

            return carry

        lax.fori_loop(0, n_outer, body, 0)
        for b in range(n_tail):
            j = n_outer * NBUF + b
            pltpu.make_async_copy(
                g_hbm.at[sidx.at[_chunk(b)]], rows.at[b], sems[b]
            ).wait()
            pltpu.sync_copy(rows.at[b], acc.at[didx.at[_chunk(j)]], add=True)
        plsc.subcore_barrier()

        @pl.when(sid < GRID)
        def _out():
            pltpu.sync_copy(
                acc.at[pl.ds(sid * stripe, stripe)],
                out_hbm.at[cid].at[pl.ds(sid * stripe, stripe)],
            )

    return kagg(g, src3, dst3, zeros_stripe)


def _dinv_block(d0_ref, d1_ref):
    return lax.rsqrt(d0_ref[...] + d1_ref[...] + 1.0)


def _tc_g1_body(x_ref, w_ref, d0_ref, d1_ref, o_ref):
    dinv = _dinv_block(d0_ref, d1_ref)
    o_ref[...] = dinv * jnp.dot(
        x_ref[...], w_ref[...], preferred_element_type=jnp.float32
    )


def _tc_mid_body(agg_ref, g_ref, d0_ref, d1_ref, b_ref, w_ref, o_ref):
    dinv = _dinv_block(d0_ref, d1_ref)
    h = dinv * (agg_ref[0] + agg_ref[1] + g_ref[...]) + b_ref[...]
    h = jnp.maximum(h, 0.0)
    o_ref[...] = dinv * jnp.dot(h, w_ref[...], preferred_element_type=jnp.float32)


def _tc_final_body(agg_ref, g_ref, d0_ref, d1_ref, b_ref, wc_ref, bc_ref, o_ref):
    dinv = _dinv_block(d0_ref, d1_ref)
    h = dinv * (agg_ref[0] + agg_ref[1] + g_ref[...]) + b_ref[...]
    o_ref[...] = (
        jnp.dot(h, wc_ref[...], preferred_element_type=jnp.float32) + bc_ref[...]
    )


def _row_spec(width):
    return pl.BlockSpec((ROW_BLK, width), lambda i: (i, 0))


_D_SPEC = pl.BlockSpec((ROW_BLK, 1), lambda i: (i, 0))
_AGG_SPEC = pl.BlockSpec((NC, ROW_BLK, H), lambda i: (0, i, 0))
_FULL_W = pl.BlockSpec((F, H), lambda i: (0, 0))
_BIAS_SPEC = pl.BlockSpec((1, H), lambda i: (0, 0))


def _tc_g1(x, W1, d0, d1):
    return pl.pallas_call(
        _tc_g1_body,
        grid=(GRID,),
        in_specs=[_row_spec(F), _FULL_W, _D_SPEC, _D_SPEC],
        out_specs=_row_spec(H),
        out_shape=jax.ShapeDtypeStruct((N, H), jnp.float32),
    )(x, W1, d0, d1)


def _tc_mid(aggp, g1, d0, d1, b1, W2):
    return pl.pallas_call(
        _tc_mid_body,
        grid=(GRID,),
        in_specs=[_AGG_SPEC, _row_spec(H), _D_SPEC, _D_SPEC, _BIAS_SPEC, _FULL_W],
        out_specs=_row_spec(H),
        out_shape=jax.ShapeDtypeStruct((N, H), jnp.float32),
    )(aggp, g1, d0, d1, b1, W2)


def _tc_final(aggp, g2, d0, d1, b2, Wc, bc):
    return pl.pallas_call(
        _tc_final_body,
        grid=(GRID,),
        in_specs=[
            _AGG_SPEC,
            _row_spec(H),
            _D_SPEC,
            _D_SPEC,
            _BIAS_SPEC,
            pl.BlockSpec((H, C), lambda i: (0, 0)),
            pl.BlockSpec((1, C), lambda i: (0, 0)),
        ],
        out_specs=pl.BlockSpec((ROW_BLK, C), lambda i: (i, 0)),
        out_shape=jax.ShapeDtypeStruct((N, C), jnp.float32),
    )(aggp, g2, d0, d1, b2, Wc, bc)


def kernel(x, edge_index, W1, b1, W2, b2, Wc, bc):
    src = jnp.asarray(edge_index[0], jnp.int32)
    dst = jnp.asarray(edge_index[1], jnp.int32)

    zeros_h = jnp.zeros((ROW_BLK, H), jnp.float32)
    zeros_hist = jnp.zeros((HROWS, 128), jnp.float32)
    idrows = jnp.arange(HROWS, dtype=jnp.int32)

    b1r = b1.reshape(1, H)
    b2r = b2.reshape(1, H)

    degp = _sc_degree(dst, idrows, zeros_hist)
    degf = degp.reshape(NC, N_PAD)[:, :N]
    d0 = degf[0].reshape(N, 1)
    d1 = degf[1].reshape(N, 1)

    g1 = _tc_g1(x, W1, d0, d1)
    agg1 = _sc_aggregate(g1, src, dst, zeros_h)
    g2 = _tc_mid(agg1, g1, d0, d1, b1r, W2)
    agg2 = _sc_aggregate(g2, src, dst, zeros_h)
    return _tc_final(agg2, g2, d0, d1, b2r, Wc, bc.reshape(1, C))
